# baseline (speedup 1.0000x reference)
import jax
import jax.numpy as jnp
from jax.experimental import pallas as pl

_NUM_RAYS = 1024
_TB = 8


def _body(idx_ref, tbl_ref, out_ld_ref, out_li_ref):
    tb = idx_ref.shape[-1]
    nl = tbl_ref.shape[0]
    idx = idx_ref[0, 0, :]
    onehot = (jax.lax.broadcasted_iota(jnp.int32, (tb, nl), 1) == idx[:, None])
    vals = jax.lax.dot_general(
        onehot.astype(jnp.float32), tbl_ref[...],
        (((1,), (0,)), ((), ())), preferred_element_type=jnp.float32)
    x = vals[:, 0:1]
    y = vals[:, 1:2]
    z = -jnp.abs(vals[:, 2:3])
    inten = jnp.abs(vals[:, 3:4])
    n = jnp.sqrt(x * x + y * y + z * z)
    ld = jnp.concatenate([x, y, z], axis=1) / jnp.maximum(n, 1e-12)
    li = jnp.broadcast_to(inten, (tb, 3))
    for r in range(tb):
        sl = pl.ds(r * _NUM_RAYS, _NUM_RAYS)
        out_ld_ref[sl, :] = jnp.broadcast_to(ld[r:r + 1, :], (_NUM_RAYS, 3))
        out_li_ref[sl, :] = jnp.broadcast_to(li[r:r + 1, :], (_NUM_RAYS, 3))


def kernel(idx, light_direction_xy, light_direction_z, light_intensity):
    b = idx.shape[0]
    tbl = jnp.concatenate(
        [light_direction_xy, light_direction_z, light_intensity], axis=1)
    grid = b // _TB
    idx3 = idx.reshape(grid, 1, _TB)
    out_shape = (b * _NUM_RAYS, 3)
    out_ld, out_li = pl.pallas_call(
        _body,
        grid=(grid,),
        in_specs=[
            pl.BlockSpec((1, 1, _TB), lambda i: (i, 0, 0)),
            pl.BlockSpec(tbl.shape, lambda i: (0, 0)),
        ],
        out_specs=[
            pl.BlockSpec((_TB * _NUM_RAYS, 3), lambda i: (i, 0)),
            pl.BlockSpec((_TB * _NUM_RAYS, 3), lambda i: (i, 0)),
        ],
        out_shape=[
            jax.ShapeDtypeStruct(out_shape, jnp.float32),
            jax.ShapeDtypeStruct(out_shape, jnp.float32),
        ],
    )(idx3, tbl)
    return (out_ld, out_li)
